# baseline (speedup 1.0000x reference)
import jax
import jax.numpy as jnp
from jax import lax
from jax.experimental import pallas as pl
from jax.experimental.pallas import tpu as pltpu
from jax.experimental.pallas import tpu_sc as plsc

B = 32
H1, H2 = 32, 8
W1, W2 = 2, 128
IN_C = 6
OUT_C = 4
NUM_CORES = 2
NUM_SUBCORES = 16
K = 4
N_CHUNKS = H1 // K
NBUF = 4


def _sc_compact(v):
    mesh = plsc.VectorSubcoreMesh(core_axis_name="c", subcore_axis_name="s")

    @pl.kernel(
        out_type=jax.ShapeDtypeStruct((B, H1, H2, W1, OUT_C, W2), jnp.float32),
        mesh=mesh,
        scratch_types=(
            [pltpu.VMEM((K, H2, W1, OUT_C, W2), jnp.float32)] * NBUF
            + [pltpu.SemaphoreType.DMA] * (2 * NBUF)
        ),
        compiler_params=pltpu.CompilerParams(
            use_tc_tiling_on_sc=False, needs_layout_passes=False),
    )
    def body(in_hbm, out_hbm, *scratch):
        cid = lax.axis_index("c")
        sid = lax.axis_index("s")
        b = sid * NUM_CORES + cid
        bufs = scratch[:NBUF]
        sem_in = scratch[NBUF:2 * NBUF]
        sem_out = scratch[2 * NBUF:]

        cps = []
        for c in range(OUT_C):
            for w1 in range(W1):
                src = in_hbm.at[b, c, :, w1]
                dst = out_hbm.at[b, :, :, w1, c, :]
                cps.append(pltpu.async_copy(src, dst, sem_in[0]))
        for cp in cps:
            cp.wait()

    return body(v)


def kernel(u_t):
    t = u_t.transpose(0, 3, 1, 2)
    t6 = t.reshape(B, IN_C, H1, H2, W1, W2)
    v = t6.transpose(0, 1, 2, 4, 3, 5)
    o6 = _sc_compact(v)
    o = o6.transpose(0, 1, 2, 3, 5, 4)
    return o.reshape(B, H1 * H2, W1 * W2, OUT_C)


# speedup vs baseline: 21.4808x; 21.4808x over previous
import jax
import jax.numpy as jnp
from jax import lax
from jax.experimental import pallas as pl
from jax.experimental.pallas import tpu as pltpu
from jax.experimental.pallas import tpu_sc as plsc

B = 32
H1, H2 = 32, 8
W1, W2 = 2, 128
IN_C = 6
OUT_C = 4
NUM_CORES = 2
NUM_SUBCORES = 16
K = 4
N_CHUNKS = H1 // K
NBUF = 4


def _sc_compact(v):
    mesh = plsc.VectorSubcoreMesh(core_axis_name="c", subcore_axis_name="s")

    @pl.kernel(
        out_type=jax.ShapeDtypeStruct((B, H1, H2, W1, OUT_C, W2), jnp.float32),
        mesh=mesh,
        scratch_types=(
            [pltpu.VMEM((K, H2, W1, OUT_C, W2), jnp.float32)] * NBUF
            + [pltpu.SemaphoreType.DMA] * (2 * NBUF)
        ),
        compiler_params=pltpu.CompilerParams(
            use_tc_tiling_on_sc=False, needs_layout_passes=False),
    )
    def body(in_hbm, out_hbm, *scratch):
        cid = lax.axis_index("c")
        sid = lax.axis_index("s")
        b = sid * NUM_CORES + cid
        bufs = scratch[:NBUF]
        sem_in = scratch[NBUF:2 * NBUF]
        sem_out = scratch[2 * NBUF:]

        def start_in(i, bu):
            cps = []
            for c in range(OUT_C):
                for w1 in range(W1):
                    src = in_hbm.at[b, c, pl.ds(i * K, K), w1]
                    dst = bufs[bu].at[:, :, w1, c, :]
                    cps.append(pltpu.async_copy(src, dst, sem_in[bu]))
            return cps

        def start_out(i, bu):
            dst = out_hbm.at[b, pl.ds(i * K, K)]
            return pltpu.async_copy(bufs[bu], dst, sem_out[bu])

        PRE = 2
        cp_in = [None] * N_CHUNKS
        cp_out = [None] * N_CHUNKS
        for j in range(min(PRE, N_CHUNKS)):
            cp_in[j] = start_in(j, j % NBUF)
        out_waited = 0
        for i in range(N_CHUNKS):
            nxt = i + PRE
            if nxt < N_CHUNKS:
                while out_waited <= nxt - NBUF:
                    cp_out[out_waited].wait()
                    out_waited += 1
                cp_in[nxt] = start_in(nxt, nxt % NBUF)
            for cp in cp_in[i]:
                cp.wait()
            cp_out[i] = start_out(i, i % NBUF)
        while out_waited < N_CHUNKS:
            cp_out[out_waited].wait()
            out_waited += 1

    return body(v)


def kernel(u_t):
    t = u_t.transpose(0, 3, 1, 2)
    t6 = t.reshape(B, IN_C, H1, H2, W1, W2)
    v = t6.transpose(0, 1, 2, 4, 3, 5)
    o6 = _sc_compact(v)
    o = o6.transpose(0, 1, 2, 3, 5, 4)
    return o.reshape(B, H1 * H2, W1 * W2, OUT_C)


# speedup vs baseline: 24.7913x; 1.1541x over previous
import jax
import jax.numpy as jnp
from jax import lax
from jax.experimental import pallas as pl
from jax.experimental.pallas import tpu as pltpu
from jax.experimental.pallas import tpu_sc as plsc

B = 32
H1, H2 = 32, 8
W1, W2 = 2, 128
IN_C = 6
OUT_C = 4
NUM_CORES = 2
NUM_SUBCORES = 16
K = 4
N_CHUNKS = H1 // K
NBUF = 4


def _sc_compact(v):
    mesh = plsc.VectorSubcoreMesh(core_axis_name="c", subcore_axis_name="s")

    @pl.kernel(
        out_type=jax.ShapeDtypeStruct((B, H1, H2, W1, OUT_C, W2), jnp.float32),
        mesh=mesh,
        scratch_types=(
            [pltpu.VMEM((K, W1, H2, OUT_C, W2), jnp.float32)] * NBUF
            + [pltpu.SemaphoreType.DMA] * (2 * NBUF)
        ),
        compiler_params=pltpu.CompilerParams(
            use_tc_tiling_on_sc=False, needs_layout_passes=False),
    )
    def body(in_hbm, out_hbm, *scratch):
        cid = lax.axis_index("c")
        sid = lax.axis_index("s")
        b = sid * NUM_CORES + cid
        bufs = scratch[:NBUF]
        sem_in = scratch[NBUF:2 * NBUF]
        sem_out = scratch[2 * NBUF:]

        def start_in(i, bu):
            cps = []
            for c in range(OUT_C):
                src = in_hbm.at[b, c, pl.ds(i * K, K)]
                dst = bufs[bu].at[:, :, :, c, :]
                cps.append(pltpu.async_copy(src, dst, sem_in[bu]))
            return cps

        def start_out(i, bu):
            cps = []
            for w1 in range(W1):
                src = bufs[bu].at[:, w1]
                dst = out_hbm.at[b, pl.ds(i * K, K), :, w1]
                cps.append(pltpu.async_copy(src, dst, sem_out[bu]))
            return cps

        PRE = 2
        cp_in = [None] * N_CHUNKS
        cp_out = [None] * N_CHUNKS
        for j in range(min(PRE, N_CHUNKS)):
            cp_in[j] = start_in(j, j % NBUF)
        out_waited = 0
        for i in range(N_CHUNKS):
            nxt = i + PRE
            if nxt < N_CHUNKS:
                while out_waited <= nxt - NBUF:
                    for cp in cp_out[out_waited]:
                        cp.wait()
                    out_waited += 1
                cp_in[nxt] = start_in(nxt, nxt % NBUF)
            for cp in cp_in[i]:
                cp.wait()
            cp_out[i] = start_out(i, i % NBUF)
        while out_waited < N_CHUNKS:
            for cp in cp_out[out_waited]:
                cp.wait()
            out_waited += 1

    return body(v)


def kernel(u_t):
    t = u_t.transpose(0, 3, 1, 2)
    t6 = t.reshape(B, IN_C, H1, H2, W1, W2)
    v = t6.transpose(0, 1, 2, 4, 3, 5)
    o6 = _sc_compact(v)
    o = o6.transpose(0, 1, 2, 3, 5, 4)
    return o.reshape(B, H1 * H2, W1 * W2, OUT_C)
